# baseline (speedup 1.0000x reference)
import functools

import jax
import jax.numpy as jnp
from jax import lax
from jax.experimental import pallas as pl
from jax.experimental.pallas import tpu as pltpu
from jax.experimental.pallas import tpu_sc as plsc

N = 10000
NPAD = 10240
E = 320000
CH = 128
NCHUNK = E // CH
D_IN = 128
D_HID = 256
HF = 128
NC = 2
NS = 16
RPS = NPAD // NS
RB = 512
GRID = NPAD // RB

_mesh = plsc.VectorSubcoreMesh(
    core_axis_name="c", subcore_axis_name="s", num_cores=NC, num_subcores=NS)


CPS = 157
CPS_PAD = 160
BP = 32
NBLK = CPS_PAD // BP


def _msg_body(s_hbm, idx_hbm, dst_hbm, zrows_hbm, acc_hbm,
              acc_sh, idxall, dstall, rows_a, rows_b,
              sem_ga, sem_gb, sem_sa, sem_sb):
    c = lax.axis_index("c")
    s = lax.axis_index("s")
    row0 = s * RPS

    pltpu.sync_copy(zrows_hbm, rows_a)
    for t in range(RPS // CH):
        pltpu.sync_copy(rows_a, acc_sh.at[pl.ds(row0 + t * CH, CH), :])

    cnt = 156 + (s < 4).astype(jnp.int32)
    bufs = ((rows_a, sem_ga, sem_sa), (rows_b, sem_gb, sem_sb))

    def gather(jj, rbuf, sem):
        pltpu.async_copy(s_hbm.at[idxall.at[jj]], rbuf, sem)

    def wait_gather(jj, rbuf, sem):
        pltpu.make_async_copy(s_hbm.at[idxall.at[jj]], rbuf, sem).wait()

    def scatter(jj, rbuf, sem):
        pltpu.async_copy(rbuf, acc_sh.at[dstall.at[jj]], sem, add=True)

    def wait_scatter(rbuf, sem):
        pltpu.make_async_copy(rbuf, acc_sh.at[dstall.at[0]], sem).wait()

    plsc.subcore_barrier()

    def blk_body(bi, carry):
        k0 = bi * BP
        nin = jnp.clip(cnt - k0, 0, BP)

        @pl.when(k0 < cnt)
        def _():
            pltpu.sync_copy(idx_hbm.at[c, s, pl.ds(k0, BP), :], idxall)
            pltpu.sync_copy(dst_hbm.at[s, pl.ds(k0, BP), :], dstall)
            gather(0, rows_a, sem_ga)

            def pair_body(g, carry2):
                for b in range(2):
                    j = 2 * g + b
                    k = k0 + j
                    rbuf, gsem, ssem = bufs[b]
                    nbuf, ngsem, nssem = bufs[1 - b]

                    @pl.when(k < cnt)
                    def _():
                        wait_gather(j, rbuf, gsem)
                        if b == 0:
                            @pl.when(j >= 1)
                            def _():
                                wait_scatter(nbuf, nssem)
                        else:
                            wait_scatter(nbuf, nssem)

                        @pl.when(jnp.logical_and(j + 1 < BP, k + 1 < cnt))
                        def _():
                            gather(j + 1, nbuf, ngsem)

                        scatter(j, rbuf, ssem)

                return carry2

            lax.fori_loop(0, BP // 2, pair_body, 0)

            @pl.when(lax.rem(nin, 2) == 1)
            def _():
                wait_scatter(rows_a, sem_sa)

            @pl.when(lax.rem(nin, 2) == 0)
            def _():
                wait_scatter(rows_b, sem_sb)

        return carry

    lax.fori_loop(0, NBLK, blk_body, 0)
    plsc.subcore_barrier()

    base_off = c * NPAD
    for t in range(RPS // CH):
        rr = row0 + t * CH
        pltpu.sync_copy(acc_sh.at[pl.ds(rr, CH), :], rows_a)
        pltpu.sync_copy(rows_a, acc_hbm.at[pl.ds(base_off + rr, CH), :])


_msg_scratch = [
    pltpu.VMEM_SHARED((NPAD, HF), jnp.float32),
    pltpu.VMEM((BP, CH), jnp.int32),
    pltpu.VMEM((BP, CH), jnp.int32),
    pltpu.VMEM((CH, HF), jnp.float32),
    pltpu.VMEM((CH, HF), jnp.float32),
    pltpu.SemaphoreType.DMA,
    pltpu.SemaphoreType.DMA,
    pltpu.SemaphoreType.DMA,
    pltpu.SemaphoreType.DMA,
]
_msg_kernel = pl.kernel(
    _msg_body,
    out_type=jax.ShapeDtypeStruct((NC * NPAD, HF), jnp.float32),
    mesh=_mesh,
    scratch_types=_msg_scratch,
)


def _tc1_body(x_ref, deg_ref, w_ref, out_ref):
    dis = lax.rsqrt(deg_ref[...] + 1.0)
    xw = jnp.dot(x_ref[...], w_ref[...], preferred_element_type=jnp.float32)
    sc = dis * xw
    out_ref[0] = sc[:, :HF]
    out_ref[1] = sc[:, HF:]


def _tc_layer_body(acc_ref, s_ref, deg_ref, w_ref, b_ref, out_ref):
    dis = lax.rsqrt(deg_ref[...] + 1.0)
    a0 = jnp.maximum(dis * (acc_ref[0] + s_ref[0]) + b_ref[0:1, :], 0.0)
    a1 = jnp.maximum(dis * (acc_ref[1] + s_ref[1]) + b_ref[1:2, :], 0.0)
    af = jnp.concatenate([a0, a1], axis=1)
    h = jnp.dot(af, w_ref[...], preferred_element_type=jnp.float32)
    sc = dis * h
    out_ref[0] = sc[:, :HF]
    out_ref[1] = sc[:, HF:]


def _tc_final_body(acc_ref, s_ref, deg_ref, b_ref, out_ref):
    dis = lax.rsqrt(deg_ref[...] + 1.0)
    out_ref[:, :HF] = dis * (acc_ref[0] + s_ref[0]) + b_ref[0:1, :]
    out_ref[:, HF:] = dis * (acc_ref[1] + s_ref[1]) + b_ref[1:2, :]


_half_spec = pl.BlockSpec((2, RB, HF), lambda i: (0, i, 0))
_deg_spec = pl.BlockSpec((RB, 1), lambda i: (i, 0))
_half_out = jax.ShapeDtypeStruct((2, NPAD, HF), jnp.float32)


def _tc1(x_pad, deg3, W1):
    return pl.pallas_call(
        _tc1_body,
        grid=(GRID,),
        in_specs=[
            pl.BlockSpec((RB, D_IN), lambda i: (i, 0)),
            _deg_spec,
            pl.BlockSpec((D_IN, D_HID), lambda i: (0, 0)),
        ],
        out_specs=_half_spec,
        out_shape=_half_out,
    )(x_pad, deg3, W1)


def _tc_layer(acc, s_prev, deg3, W, bh):
    return pl.pallas_call(
        _tc_layer_body,
        grid=(GRID,),
        in_specs=[
            _half_spec,
            _half_spec,
            _deg_spec,
            pl.BlockSpec((D_HID, D_HID), lambda i: (0, 0)),
            pl.BlockSpec((2, HF), lambda i: (0, 0)),
        ],
        out_specs=_half_spec,
        out_shape=_half_out,
    )(acc, s_prev, deg3, W, bh)


def _tc_final(acc, s_prev, deg3, bh):
    return pl.pallas_call(
        _tc_final_body,
        grid=(GRID,),
        in_specs=[
            _half_spec,
            _half_spec,
            _deg_spec,
            pl.BlockSpec((2, HF), lambda i: (0, 0)),
        ],
        out_specs=pl.BlockSpec((RB, D_HID), lambda i: (i, 0)),
        out_shape=jax.ShapeDtypeStruct((NPAD, D_HID), jnp.float32),
    )(acc, s_prev, deg3, bh)


def kernel(x, edge_index, W1, b1, W2, b2, W3, b3):
    src = edge_index[0]
    dst = edge_index[1]
    x_pad = jnp.pad(x, ((0, NPAD - N), (0, 0)))
    z_rows = jnp.zeros((CH, HF), jnp.float32)

    src2d = src.reshape(NCHUNK, CH)
    dst2d = dst.reshape(NCHUNK, CH)
    starts = jnp.arange(NS) * 156 + jnp.minimum(jnp.arange(NS), 4)
    row_ids = jnp.clip(starts[:, None] + jnp.arange(CPS_PAD)[None, :], 0, NCHUNK - 1)
    src3d = src2d[row_ids]
    dst3d = dst2d[row_ids]
    idx3d = jnp.stack([src3d, src3d + NPAD])

    ones_tab = jnp.ones((NC * NPAD, HF), jnp.float32)
    degacc = _msg_kernel(ones_tab, idx3d, dst3d, z_rows)
    deg3 = degacc[:NPAD, 0:1]

    b1h = b1.reshape(2, HF)
    b2h = b2.reshape(2, HF)
    b3h = b3.reshape(2, HF)

    s1 = _tc1(x_pad, deg3, W1)
    acc1 = _msg_kernel(s1.reshape(NC * NPAD, HF), idx3d, dst3d, z_rows).reshape(2, NPAD, HF)
    s2 = _tc_layer(acc1, s1, deg3, W2, b1h)
    acc2 = _msg_kernel(s2.reshape(NC * NPAD, HF), idx3d, dst3d, z_rows).reshape(2, NPAD, HF)
    s3 = _tc_layer(acc2, s2, deg3, W3, b2h)
    acc3 = _msg_kernel(s3.reshape(NC * NPAD, HF), idx3d, dst3d, z_rows).reshape(2, NPAD, HF)
    out = _tc_final(acc3, s3, deg3, b3h)
    return out[:N]


# speedup vs baseline: 1.2836x; 1.2836x over previous
import functools

import jax
import jax.numpy as jnp
from jax import lax
from jax.experimental import pallas as pl
from jax.experimental.pallas import tpu as pltpu
from jax.experimental.pallas import tpu_sc as plsc

N = 10000
NPAD = 10240
E = 320000
CH = 128
NCHUNK = E // CH
D_IN = 128
D_HID = 256
HF = 128
NC = 2
NS = 16
RPS = NPAD // NS
RB = 512
GRID = NPAD // RB

_mesh = plsc.VectorSubcoreMesh(
    core_axis_name="c", subcore_axis_name="s", num_cores=NC, num_subcores=NS)


CPS = 157
CPS_PAD = 160
BP = 32
NBLK = CPS_PAD // BP


def _msg_body(s_hbm, idx_hbm, dst_hbm, zrows_hbm, acc_hbm,
              acc_sh, idxall, dstall, rows_a, rows_b,
              sem_ga, sem_gb, sem_sa, sem_sb):
    c = lax.axis_index("c")
    s = lax.axis_index("s")
    row0 = s * RPS

    pltpu.sync_copy(zrows_hbm, rows_a)
    for t in range(RPS // CH):
        pltpu.sync_copy(rows_a, acc_sh.at[pl.ds(row0 + t * CH, CH), :])

    cnt = 156 + (s < 4).astype(jnp.int32)
    bufs = ((rows_a, sem_ga, sem_sa), (rows_b, sem_gb, sem_sb))

    def gather(jj, rbuf, sem):
        pltpu.async_copy(s_hbm.at[idxall.at[jj]], rbuf, sem)

    def wait_gather(jj, rbuf, sem):
        pltpu.make_async_copy(s_hbm.at[idxall.at[jj]], rbuf, sem).wait()

    def scatter(jj, rbuf, sem):
        pltpu.async_copy(rbuf, acc_sh.at[dstall.at[jj]], sem, add=True)

    def wait_scatter(rbuf, sem):
        pltpu.make_async_copy(rbuf, acc_sh.at[dstall.at[0]], sem).wait()

    plsc.subcore_barrier()

    def blk_body(bi, carry):
        k0 = bi * BP
        nin = jnp.clip(cnt - k0, 0, BP)

        @pl.when(k0 < cnt)
        def _():
            pltpu.sync_copy(idx_hbm.at[c, s, pl.ds(k0, BP), :], idxall)
            pltpu.sync_copy(dst_hbm.at[s, pl.ds(k0, BP), :], dstall)
            gather(0, rows_a, sem_ga)

            def pair_body(g, carry2):
                for b in range(2):
                    j = 2 * g + b
                    k = k0 + j
                    rbuf, gsem, ssem = bufs[b]
                    nbuf, ngsem, nssem = bufs[1 - b]

                    @pl.when(k < cnt)
                    def _():
                        if b == 0:
                            @pl.when(j >= 1)
                            def _():
                                wait_scatter(nbuf, nssem)
                        else:
                            wait_scatter(nbuf, nssem)

                        @pl.when(jnp.logical_and(j + 1 < BP, k + 1 < cnt))
                        def _():
                            gather(j + 1, nbuf, ngsem)

                        wait_gather(j, rbuf, gsem)
                        scatter(j, rbuf, ssem)

                return carry2

            lax.fori_loop(0, BP // 2, pair_body, 0)

            @pl.when(lax.rem(nin, 2) == 1)
            def _():
                wait_scatter(rows_a, sem_sa)

            @pl.when(lax.rem(nin, 2) == 0)
            def _():
                wait_scatter(rows_b, sem_sb)

        return carry

    lax.fori_loop(0, NBLK, blk_body, 0)
    plsc.subcore_barrier()

    base_off = c * NPAD
    for t in range(RPS // CH):
        rr = row0 + t * CH
        pltpu.sync_copy(acc_sh.at[pl.ds(rr, CH), :], rows_a)
        pltpu.sync_copy(rows_a, acc_hbm.at[pl.ds(base_off + rr, CH), :])


_msg_scratch = [
    pltpu.VMEM_SHARED((NPAD, HF), jnp.float32),
    pltpu.VMEM((BP, CH), jnp.int32),
    pltpu.VMEM((BP, CH), jnp.int32),
    pltpu.VMEM((CH, HF), jnp.float32),
    pltpu.VMEM((CH, HF), jnp.float32),
    pltpu.SemaphoreType.DMA,
    pltpu.SemaphoreType.DMA,
    pltpu.SemaphoreType.DMA,
    pltpu.SemaphoreType.DMA,
]
_msg_kernel = pl.kernel(
    _msg_body,
    out_type=jax.ShapeDtypeStruct((NC * NPAD, HF), jnp.float32),
    mesh=_mesh,
    scratch_types=_msg_scratch,
)


def _make_deg_kernel(dw):
    def body(dst_hbm, ones_hbm, zcol_hbm, deg_hbm,
             acc_sh, dstall, onesbuf, obuf, sem):
        c = lax.axis_index("c")
        s = lax.axis_index("s")
        row0 = s * RPS

        pltpu.sync_copy(ones_hbm, onesbuf)
        pltpu.sync_copy(zcol_hbm, obuf)
        for t in range(RPS // CH):
            pltpu.sync_copy(obuf, acc_sh.at[pl.ds(row0 + t * CH, CH), :])

        cnt = 156 + (s < 4).astype(jnp.int32)
        plsc.subcore_barrier()

        def blk_body(bi, carry):
            k0 = bi * BP

            @pl.when(k0 < cnt)
            def _():
                pltpu.sync_copy(dst_hbm.at[s, pl.ds(k0, BP), :], dstall)

                def issue(j, carry2):
                    @pl.when(k0 + j < cnt)
                    def _():
                        pltpu.async_copy(
                            onesbuf, acc_sh.at[dstall.at[j]], sem, add=True)
                    return carry2

                lax.fori_loop(0, BP, issue, 0)

                def drain(j, carry2):
                    @pl.when(k0 + j < cnt)
                    def _():
                        pltpu.make_async_copy(
                            onesbuf, acc_sh.at[dstall.at[0]], sem).wait()
                    return carry2

                lax.fori_loop(0, BP, drain, 0)

            return carry

        lax.fori_loop(0, NBLK, blk_body, 0)
        plsc.subcore_barrier()

        for t in range(RPS // CH):
            rr = row0 + t * CH
            pltpu.sync_copy(acc_sh.at[pl.ds(rr, CH), :], obuf)
            pltpu.sync_copy(obuf, deg_hbm.at[pl.ds(c * NPAD + rr, CH), :])

    scratch = [
        pltpu.VMEM_SHARED((NPAD, dw), jnp.float32),
        pltpu.VMEM((BP, CH), jnp.int32),
        pltpu.VMEM((CH, dw), jnp.float32),
        pltpu.VMEM((CH, dw), jnp.float32),
        pltpu.SemaphoreType.DMA,
    ]
    return body, scratch


DW = 128
_deg_body, _deg_scratch = _make_deg_kernel(DW)
_deg_kernel = pl.kernel(
    _deg_body,
    out_type=jax.ShapeDtypeStruct((NC * NPAD, DW), jnp.float32),
    mesh=_mesh,
    scratch_types=_deg_scratch,
)


def _tc1_body(x_ref, deg_ref, w_ref, out_ref):
    dis = lax.rsqrt(deg_ref[...] + 1.0)
    xw = jnp.dot(x_ref[...], w_ref[...], preferred_element_type=jnp.float32)
    sc = dis * xw
    out_ref[0] = sc[:, :HF]
    out_ref[1] = sc[:, HF:]


def _tc_layer_body(acc_ref, s_ref, deg_ref, w_ref, b_ref, out_ref):
    dis = lax.rsqrt(deg_ref[...] + 1.0)
    a0 = jnp.maximum(dis * (acc_ref[0] + s_ref[0]) + b_ref[0:1, :], 0.0)
    a1 = jnp.maximum(dis * (acc_ref[1] + s_ref[1]) + b_ref[1:2, :], 0.0)
    af = jnp.concatenate([a0, a1], axis=1)
    h = jnp.dot(af, w_ref[...], preferred_element_type=jnp.float32)
    sc = dis * h
    out_ref[0] = sc[:, :HF]
    out_ref[1] = sc[:, HF:]


def _tc_final_body(acc_ref, s_ref, deg_ref, b_ref, out_ref):
    dis = lax.rsqrt(deg_ref[...] + 1.0)
    out_ref[:, :HF] = dis * (acc_ref[0] + s_ref[0]) + b_ref[0:1, :]
    out_ref[:, HF:] = dis * (acc_ref[1] + s_ref[1]) + b_ref[1:2, :]


_half_spec = pl.BlockSpec((2, RB, HF), lambda i: (0, i, 0))
_deg_spec = pl.BlockSpec((RB, 1), lambda i: (i, 0))
_half_out = jax.ShapeDtypeStruct((2, NPAD, HF), jnp.float32)


def _tc1(x_pad, deg3, W1):
    return pl.pallas_call(
        _tc1_body,
        grid=(GRID,),
        in_specs=[
            pl.BlockSpec((RB, D_IN), lambda i: (i, 0)),
            _deg_spec,
            pl.BlockSpec((D_IN, D_HID), lambda i: (0, 0)),
        ],
        out_specs=_half_spec,
        out_shape=_half_out,
    )(x_pad, deg3, W1)


def _tc_layer(acc, s_prev, deg3, W, bh):
    return pl.pallas_call(
        _tc_layer_body,
        grid=(GRID,),
        in_specs=[
            _half_spec,
            _half_spec,
            _deg_spec,
            pl.BlockSpec((D_HID, D_HID), lambda i: (0, 0)),
            pl.BlockSpec((2, HF), lambda i: (0, 0)),
        ],
        out_specs=_half_spec,
        out_shape=_half_out,
    )(acc, s_prev, deg3, W, bh)


def _tc_final(acc, s_prev, deg3, bh):
    return pl.pallas_call(
        _tc_final_body,
        grid=(GRID,),
        in_specs=[
            _half_spec,
            _half_spec,
            _deg_spec,
            pl.BlockSpec((2, HF), lambda i: (0, 0)),
        ],
        out_specs=pl.BlockSpec((RB, D_HID), lambda i: (i, 0)),
        out_shape=jax.ShapeDtypeStruct((NPAD, D_HID), jnp.float32),
    )(acc, s_prev, deg3, bh)


def kernel(x, edge_index, W1, b1, W2, b2, W3, b3):
    src = edge_index[0]
    dst = edge_index[1]
    x_pad = jnp.pad(x, ((0, NPAD - N), (0, 0)))
    z_rows = jnp.zeros((CH, HF), jnp.float32)

    src2d = src.reshape(NCHUNK, CH)
    dst2d = dst.reshape(NCHUNK, CH)
    starts = jnp.arange(NS) * 156 + jnp.minimum(jnp.arange(NS), 4)
    row_ids = jnp.clip(starts[:, None] + jnp.arange(CPS_PAD)[None, :], 0, NCHUNK - 1)
    src3d = src2d[row_ids]
    dst3d = dst2d[row_ids]
    idx3d = jnp.stack([src3d, src3d + NPAD])

    ones_deg = jnp.ones((CH, DW), jnp.float32)
    z_deg = jnp.zeros((CH, DW), jnp.float32)
    degacc = _deg_kernel(dst3d, ones_deg, z_deg)
    deg3 = degacc[:NPAD, 0:1]

    b1h = b1.reshape(2, HF)
    b2h = b2.reshape(2, HF)
    b3h = b3.reshape(2, HF)

    s1 = _tc1(x_pad, deg3, W1)
    acc1 = _msg_kernel(s1.reshape(NC * NPAD, HF), idx3d, dst3d, z_rows).reshape(2, NPAD, HF)
    s2 = _tc_layer(acc1, s1, deg3, W2, b1h)
    acc2 = _msg_kernel(s2.reshape(NC * NPAD, HF), idx3d, dst3d, z_rows).reshape(2, NPAD, HF)
    s3 = _tc_layer(acc2, s2, deg3, W3, b2h)
    acc3 = _msg_kernel(s3.reshape(NC * NPAD, HF), idx3d, dst3d, z_rows).reshape(2, NPAD, HF)
    out = _tc_final(acc3, s3, deg3, b3h)
    return out[:N]
